# baseline (speedup 1.0000x reference)
import jax
import jax.numpy as jnp
from jax.experimental import pallas as pl
from jax.experimental.pallas import tpu as pltpu

_EPB = 4


def _moe_body(x_ref, w1_ref, w2_ref, tw_ref, ids_ref, out_ref):
    g = pl.program_id(0)
    n = w2_ref.shape[2]
    x = x_ref[...]
    dn = (((1,), (1,)), ((), ()))
    for i in range(_EPB):
        e = g * _EPB + i
        h = jax.lax.dot_general(x, w1_ref[i], dn,
                                preferred_element_type=jnp.float32)
        gate = h[:, :n]
        up = h[:, n:]
        act = gate * jax.nn.sigmoid(gate) * up
        y = jax.lax.dot_general(act, w2_ref[i], dn,
                                preferred_element_type=jnp.float32)
        sel = (ids_ref[...] == e).astype(jnp.float32)
        wpe = jnp.sum(tw_ref[...] * sel, axis=1, keepdims=True)
        contrib = wpe * y
        if i == 0:
            @pl.when(g == 0)
            def _init():
                out_ref[...] = contrib

            @pl.when(g > 0)
            def _acc():
                out_ref[...] += contrib
        else:
            out_ref[...] += contrib


def kernel(hidden_states, w1, w2, topk_weights, topk_ids):
    m, k = hidden_states.shape
    e_total, two_n, _ = w1.shape
    n = w2.shape[2]
    topk = topk_ids.shape[1]
    return pl.pallas_call(
        _moe_body,
        grid=(e_total // _EPB,),
        in_specs=[
            pl.BlockSpec((m, k), lambda g: (0, 0)),
            pl.BlockSpec((_EPB, two_n, k), lambda g: (g, 0, 0)),
            pl.BlockSpec((_EPB, k, n), lambda g: (g, 0, 0)),
            pl.BlockSpec((m, topk), lambda g: (0, 0)),
            pl.BlockSpec((m, topk), lambda g: (0, 0)),
        ],
        out_specs=pl.BlockSpec((m, k), lambda g: (0, 0)),
        out_shape=jax.ShapeDtypeStruct((m, k), jnp.float32),
        compiler_params=pltpu.CompilerParams(
            dimension_semantics=("arbitrary",)),
    )(hidden_states, w1, w2, topk_weights, topk_ids)


# speedup vs baseline: 1.0440x; 1.0440x over previous
import jax
import jax.numpy as jnp
from jax.experimental import pallas as pl
from jax.experimental.pallas import tpu as pltpu

_EPB = 2


def _moe_body(x_ref, w1g_ref, w1u_ref, w2a_ref, w2b_ref, tw_ref, ids_ref,
              out_ref):
    g = pl.program_id(0)
    x = x_ref[...]
    dn = (((1,), (1,)), ((), ()))
    kh = w2a_ref.shape[2]
    for i in range(_EPB):
        e = g * _EPB + i
        gate = jax.lax.dot_general(x, w1g_ref[i, 0], dn,
                                   preferred_element_type=jnp.float32)
        up = jax.lax.dot_general(x, w1u_ref[i, 0], dn,
                                 preferred_element_type=jnp.float32)
        act = gate * jax.nn.sigmoid(gate) * up
        ya = jax.lax.dot_general(act, w2a_ref[i, 0], dn,
                                 preferred_element_type=jnp.float32)
        yb = jax.lax.dot_general(act, w2b_ref[i, 0], dn,
                                 preferred_element_type=jnp.float32)
        sel = (ids_ref[...] == e).astype(jnp.float32)
        wpe = jnp.sum(tw_ref[...] * sel, axis=1, keepdims=True)
        if i == 0:
            @pl.when(g == 0)
            def _init():
                out_ref[:, :kh] = wpe * ya
                out_ref[:, kh:] = wpe * yb

            @pl.when(g > 0)
            def _acc():
                out_ref[:, :kh] += wpe * ya
                out_ref[:, kh:] += wpe * yb
        else:
            out_ref[:, :kh] += wpe * ya
            out_ref[:, kh:] += wpe * yb


def kernel(hidden_states, w1, w2, topk_weights, topk_ids):
    m, k = hidden_states.shape
    e_total, two_n, _ = w1.shape
    n = w2.shape[2]
    topk = topk_ids.shape[1]
    kh = k // 2
    w1r = w1.reshape(e_total, 2, n, k)
    w2r = w2.reshape(e_total, 2, kh, n)
    return pl.pallas_call(
        _moe_body,
        grid=(e_total // _EPB,),
        in_specs=[
            pl.BlockSpec((m, k), lambda g: (0, 0)),
            pl.BlockSpec((_EPB, 1, n, k), lambda g: (g, 0, 0, 0)),
            pl.BlockSpec((_EPB, 1, n, k), lambda g: (g, 1, 0, 0)),
            pl.BlockSpec((_EPB, 1, kh, n), lambda g: (g, 0, 0, 0)),
            pl.BlockSpec((_EPB, 1, kh, n), lambda g: (g, 1, 0, 0)),
            pl.BlockSpec((m, topk), lambda g: (0, 0)),
            pl.BlockSpec((m, topk), lambda g: (0, 0)),
        ],
        out_specs=pl.BlockSpec((m, k), lambda g: (0, 0)),
        out_shape=jax.ShapeDtypeStruct((m, k), jnp.float32),
        compiler_params=pltpu.CompilerParams(
            dimension_semantics=("arbitrary",)),
    )(hidden_states, w1r, w1r, w2r, w2r, topk_weights, topk_ids)
